# baseline (speedup 1.0000x reference)
import jax
import jax.numpy as jnp
from jax.experimental import pallas as pl
from jax.experimental.pallas import tpu as pltpu

E = 64
K = 8
G = 8
KG = 3
GS = E // G
NK = 2


def _gate_kernel(h_ref, w_ref, idx_ref, wgt_ref, acc_ref):
    k = pl.program_id(1)
    h = h_ref[...]
    w = w_ref[...]
    part = jax.lax.dot_general(
        w, h, (((1,), (1,)), ((), ())),
        preferred_element_type=jnp.float32)

    @pl.when(k == 0)
    def _init():
        acc_ref[...] = part

    @pl.when(k > 0)
    def _acc():
        acc_ref[...] += part

    @pl.when(k == NK - 1)
    def _epilogue():
        logits = acc_ref[...]
        B = logits.shape[1]

        m = jnp.max(logits, axis=0, keepdims=True)
        e = jnp.exp(logits - m)

        ge = jnp.max(e.reshape(G, GS, B), axis=1)

        grows = jax.lax.broadcasted_iota(
            jnp.int32, ge.shape, 0).astype(jnp.float32)
        gsel = jnp.zeros_like(ge)
        for _ in range(KG):
            gmv = jnp.max(ge, axis=0, keepdims=True)
            gamax = jnp.min(jnp.where(ge == gmv, grows, float(G)),
                            axis=0, keepdims=True)
            hit = grows == gamax
            gsel = jnp.where(hit, 1.0, gsel)
            ge = jnp.where(hit, -1.0, ge)

        onehot = (jax.lax.broadcasted_iota(jnp.int32, (E, G), 0) // GS ==
                  jax.lax.broadcasted_iota(jnp.int32, (E, G), 1)
                  ).astype(jnp.float32)
        emask = jax.lax.dot_general(
            onehot, gsel, (((1,), (0,)), ((), ())),
            preferred_element_type=jnp.float32)
        cur = jnp.where(emask == 1.0, e, 0.0)

        rows = jax.lax.broadcasted_iota(
            jnp.int32, cur.shape, 0).astype(jnp.float32)
        idxs, vals = [], []
        for _ in range(K):
            mv = jnp.max(cur, axis=0, keepdims=True)
            amax = jnp.min(jnp.where(cur == mv, rows, float(E)),
                           axis=0, keepdims=True)
            idxs.append(amax)
            vals.append(mv)
            cur = jnp.where(rows == amax, -1.0, cur)
        vals = jnp.concatenate(vals, axis=0)
        idxs_f = jnp.concatenate(idxs, axis=0)
        denom = jnp.sum(vals, axis=0, keepdims=True)
        wgt = vals / denom
        idx_ref[...] = idxs_f.T.astype(jnp.int32)
        wgt_ref[...] = wgt.T


def kernel(hidden_states, weight):
    T, D = hidden_states.shape
    B = 2048
    DK = D // NK
    grid = (T // B, NK)
    idx, wgt = pl.pallas_call(
        _gate_kernel,
        grid=grid,
        compiler_params=pltpu.CompilerParams(
            dimension_semantics=("parallel", "arbitrary")),
        in_specs=[
            pl.BlockSpec((B, DK), lambda i, k: (i, k)),
            pl.BlockSpec((E, DK), lambda i, k: (0, k)),
        ],
        out_specs=[
            pl.BlockSpec((B, K), lambda i, k: (i, 0)),
            pl.BlockSpec((B, K), lambda i, k: (i, 0)),
        ],
        out_shape=[
            jax.ShapeDtypeStruct((T, K), jnp.int32),
            jax.ShapeDtypeStruct((T, K), jnp.float32),
        ],
        scratch_shapes=[pltpu.VMEM((E, B), jnp.float32)],
    )(hidden_states, weight)
    return idx, wgt


# speedup vs baseline: 1.1833x; 1.1833x over previous
import jax
import jax.numpy as jnp
from jax.experimental import pallas as pl
from jax.experimental.pallas import tpu as pltpu

E = 64
K = 8
G = 8
KG = 3
GS = E // G


def _gate_kernel(h_ref, w_ref, idx_ref, wgt_ref):
    h = h_ref[...]
    w = w_ref[...]
    logits = jax.lax.dot_general(
        w, h, (((1,), (1,)), ((), ())),
        preferred_element_type=jnp.float32)
    B = logits.shape[1]

    m = jnp.max(logits, axis=0, keepdims=True)
    e = jnp.exp(logits - m)

    ge = jnp.max(e.reshape(G, GS, B), axis=1)

    grows = jax.lax.broadcasted_iota(jnp.int32, ge.shape, 0).astype(jnp.float32)
    gsel = jnp.zeros_like(ge)
    for _ in range(KG):
        gmv = jnp.max(ge, axis=0, keepdims=True)
        gamax = jnp.min(jnp.where(ge == gmv, grows, float(G)),
                        axis=0, keepdims=True)
        hit = grows == gamax
        gsel = jnp.where(hit, 1.0, gsel)
        ge = jnp.where(hit, -1.0, ge)

    onehot = (jax.lax.broadcasted_iota(jnp.int32, (E, G), 0) // GS ==
              jax.lax.broadcasted_iota(jnp.int32, (E, G), 1)).astype(jnp.float32)
    emask = jax.lax.dot_general(
        onehot, gsel, (((1,), (0,)), ((), ())),
        preferred_element_type=jnp.float32)
    cur = jnp.where(emask == 1.0, e, 0.0)

    rows = jax.lax.broadcasted_iota(jnp.int32, cur.shape, 0).astype(jnp.float32)
    idxs, vals = [], []
    for _ in range(K):
        mv = jnp.max(cur, axis=0, keepdims=True)
        amax = jnp.min(jnp.where(cur == mv, rows, float(E)),
                       axis=0, keepdims=True)
        idxs.append(amax)
        vals.append(mv)
        cur = jnp.where(rows == amax, -1.0, cur)
    vals = jnp.concatenate(vals, axis=0)
    idxs_f = jnp.concatenate(idxs, axis=0)
    denom = jnp.sum(vals, axis=0, keepdims=True)
    wgt = vals / denom
    idx_ref[...] = idxs_f.T.astype(jnp.int32)
    wgt_ref[...] = wgt.T


def kernel(hidden_states, weight):
    T, D = hidden_states.shape
    B = 2048
    grid = (T // B,)
    idx, wgt = pl.pallas_call(
        _gate_kernel,
        grid=grid,
        compiler_params=pltpu.CompilerParams(
            dimension_semantics=("parallel",)),
        in_specs=[
            pl.BlockSpec((B, D), lambda i: (i, 0)),
            pl.BlockSpec((E, D), lambda i: (0, 0)),
        ],
        out_specs=[
            pl.BlockSpec((B, K), lambda i: (i, 0)),
            pl.BlockSpec((B, K), lambda i: (i, 0)),
        ],
        out_shape=[
            jax.ShapeDtypeStruct((T, K), jnp.int32),
            jax.ShapeDtypeStruct((T, K), jnp.float32),
        ],
    )(hidden_states, weight)
    return idx, wgt
